# baseline (speedup 1.0000x reference)
import jax
import jax.numpy as jnp
from jax import lax
from jax.experimental import pallas as pl
from jax.experimental.pallas import tpu as pltpu
from jax.experimental.pallas import tpu_sc as plsc

_ROWS = 16384
_COLS = 2048
_NW = 32
_SPLIT = 10752
_RPW = _SPLIT // _NW
_K = 8
_NBUF = 3
_NVREG = _COLS // 16
_UNROLL = 8
_TC_BLK = 256
_TC_GRID = (_ROWS - _SPLIT) // _TC_BLK
_EPS = 1e-6


def _sc_body(mask_hbm, yh_hbm, y_hbm, out_hbm,
             mask_v, idx_v, yh_v, y_v, part_v, hsem, ysem):
    wid = lax.axis_index("s") * 2 + lax.axis_index("c")
    base = wid * _RPW

    pltpu.sync_copy(mask_hbm.at[pl.ds(base, _RPW)], mask_v)

    lane = lax.iota(jnp.int32, 16)

    def _compact(j, cnt):
        m = mask_v[pl.ds(j * 16, 16)] != 0
        rows = base + j * 16 + lane
        mi = m.astype(jnp.int32)
        pos = cnt + plsc.cumsum(mi) - 1
        plsc.store_scatter(idx_v, [pos], rows, mask=m)
        return cnt + jnp.sum(mi)

    n = lax.fori_loop(0, _RPW // 16, _compact, jnp.int32(0))
    idx_v[pl.ds(n, 16)] = jnp.full((16,), base, jnp.int32)

    nchunks = (n + _K - 1) // _K

    def _issue(c):
        p = c % _NBUF
        off = c * _K
        pltpu.async_copy(yh_hbm.at[idx_v.at[pl.ds(off, _K)]], yh_v.at[p],
                         hsem.at[p])
        pltpu.async_copy(y_hbm.at[idx_v.at[pl.ds(off, _K)]], y_v.at[p],
                         ysem.at[p])

    for w in range(_NBUF - 1):
        @pl.when(nchunks > w)
        def _prime(w=w):
            _issue(w)

    def _chunk(c, carry):
        p = c % _NBUF

        @pl.when(c + _NBUF - 1 < nchunks)
        def _next():
            _issue(c + _NBUF - 1)

        pltpu.make_async_copy(yh_hbm.at[idx_v.at[pl.ds(0, _K)]], yh_v.at[p],
                              hsem.at[p]).wait()
        pltpu.make_async_copy(y_hbm.at[idx_v.at[pl.ds(0, _K)]], y_v.at[p],
                              ysem.at[p]).wait()

        valid = jnp.minimum(n - c * _K, _K)

        def _row(r, rc):
            @plsc.parallel_loop(0, _NVREG, 1, unroll=_UNROLL, carry=rc)
            def _vblk(j, c2):
                a, cnt = c2
                yv = y_v[p, r, pl.ds(j * 16, 16)]
                hv = yh_v[p, r, pl.ds(j * 16, 16)]
                m = yv != 0.0
                d = jnp.where(m, hv - yv, 0.0)
                return a + d * d, cnt + plsc.all_reduce_population_count(m)

            return _vblk

        return lax.fori_loop(0, valid, _row, carry)

    acc, cf = lax.fori_loop(
        0, nchunks, _chunk,
        (jnp.zeros((16,), jnp.float32), jnp.zeros((16,), jnp.int32)))

    part_v[0, :] = acc
    part_v[1, :] = cf.astype(jnp.float32)
    pltpu.sync_copy(part_v, out_hbm.at[wid])


def _tc_dense(mask_ref, yh_ref, y_ref, out_ref, acc_ref, cnt_ref):
    i = pl.program_id(0)

    @pl.when(i == 0)
    def _init():
        acc_ref[0, 0] = 0.0
        cnt_ref[0, 0] = 0.0

    yh = yh_ref[...]
    yy = y_ref[...]
    w = mask_ref[0]
    nz = yy != 0.0
    d = yh - yy
    rowsum = jnp.sum(jnp.where(nz, d * d, 0.0), axis=1, keepdims=True)
    rowcnt = jnp.sum(jnp.where(nz, 1.0, 0.0), axis=1, keepdims=True)
    dims = (((1,), (0,)), ((), ()))
    acc_ref[0, 0] += jnp.sum(
        lax.dot_general(w, rowsum, dims, preferred_element_type=jnp.float32))
    cnt_ref[0, 0] += jnp.sum(
        lax.dot_general(w, rowcnt, dims, preferred_element_type=jnp.float32))

    @pl.when(i == _TC_GRID - 1)
    def _fin():
        out_ref[0, 0] = acc_ref[0, 0]
        out_ref[0, 1] = cnt_ref[0, 0]


def _tc_final(sc_ref, tc_ref, out_ref):
    p = sc_ref[...]
    acc = jnp.sum(p[:, 0, :]) + tc_ref[0, 0]
    cnt = jnp.sum(p[:, 1, :]) / 16.0 + tc_ref[0, 1]
    out_ref[0, 0] = jnp.sqrt(acc / cnt + _EPS)


def kernel(yhat, y, batch_mask):
    mask_i = batch_mask[:_SPLIT].astype(jnp.int32)
    maskf = batch_mask.astype(jnp.float32).reshape(_ROWS // _TC_BLK, 1, _TC_BLK)
    mesh = plsc.VectorSubcoreMesh(core_axis_name="c", subcore_axis_name="s")
    sc = pl.kernel(
        _sc_body,
        mesh=mesh,
        compiler_params=pltpu.CompilerParams(needs_layout_passes=False),
        out_type=jax.ShapeDtypeStruct((_NW, 2, 16), jnp.float32),
        scratch_types=[
            pltpu.VMEM((_RPW,), jnp.int32),
            pltpu.VMEM((_RPW + 16,), jnp.int32),
            pltpu.VMEM((_NBUF, _K, _COLS), jnp.float32),
            pltpu.VMEM((_NBUF, _K, _COLS), jnp.float32),
            pltpu.VMEM((2, 16), jnp.float32),
            pltpu.SemaphoreType.DMA((_NBUF,)),
            pltpu.SemaphoreType.DMA((_NBUF,)),
        ],
    )
    sc_parts = sc(mask_i, yhat, y)

    base_blk = _SPLIT // _TC_BLK
    tc_parts = pl.pallas_call(
        _tc_dense,
        grid=(_TC_GRID,),
        compiler_params=pltpu.CompilerParams(skip_device_barrier=True),
        in_specs=[
            pl.BlockSpec((1, 1, _TC_BLK), lambda i: (base_blk + i, 0, 0)),
            pl.BlockSpec((_TC_BLK, _COLS), lambda i: (base_blk + i, 0)),
            pl.BlockSpec((_TC_BLK, _COLS), lambda i: (base_blk + i, 0)),
        ],
        out_specs=pl.BlockSpec(memory_space=pltpu.SMEM),
        out_shape=jax.ShapeDtypeStruct((1, 2), jnp.float32),
        scratch_shapes=[
            pltpu.SMEM((1, 1), jnp.float32),
            pltpu.SMEM((1, 1), jnp.float32),
        ],
    )(maskf, yhat, y)

    out = pl.pallas_call(
        _tc_final,
        in_specs=[
            pl.BlockSpec(memory_space=pltpu.VMEM),
            pl.BlockSpec(memory_space=pltpu.SMEM),
        ],
        out_specs=pl.BlockSpec(memory_space=pltpu.SMEM),
        out_shape=jax.ShapeDtypeStruct((1, 1), jnp.float32),
    )(sc_parts, tc_parts)
    return out.reshape(())


# speedup vs baseline: 1.0893x; 1.0893x over previous
import jax
import jax.numpy as jnp
from jax import lax
from jax.experimental import pallas as pl
from jax.experimental.pallas import tpu as pltpu
from jax.experimental.pallas import tpu_sc as plsc

_ROWS = 16384
_COLS = 2048
_NW = 32
_SPLIT = 10240
_RPW = _SPLIT // _NW
_K = 8
_NBUF = 3
_NVREG = _COLS // 16
_UNROLL = 8
_TC_BLK = 256
_TC_GRID = (_ROWS - _SPLIT) // _TC_BLK
_EPS = 1e-6


def _sc_body(mask_hbm, yh_hbm, y_hbm, out_hbm,
             mask_v, idx_v, yh_v, y_v, part_v, hsem, ysem):
    wid = lax.axis_index("s") * 2 + lax.axis_index("c")
    base = wid * _RPW

    pltpu.sync_copy(mask_hbm.at[pl.ds(base, _RPW)], mask_v)

    lane = lax.iota(jnp.int32, 16)

    def _compact(j, cnt):
        m = mask_v[pl.ds(j * 16, 16)] != 0
        rows = base + j * 16 + lane
        mi = m.astype(jnp.int32)
        pos = cnt + plsc.cumsum(mi) - 1
        plsc.store_scatter(idx_v, [pos], rows, mask=m)
        return cnt + jnp.sum(mi)

    n = lax.fori_loop(0, _RPW // 16, _compact, jnp.int32(0))
    idx_v[pl.ds(n, 16)] = jnp.full((16,), base, jnp.int32)

    nchunks = (n + _K - 1) // _K

    def _issue(c):
        p = c % _NBUF
        off = c * _K
        pltpu.async_copy(yh_hbm.at[idx_v.at[pl.ds(off, _K)]], yh_v.at[p],
                         hsem.at[p])
        pltpu.async_copy(y_hbm.at[idx_v.at[pl.ds(off, _K)]], y_v.at[p],
                         ysem.at[p])

    for w in range(_NBUF - 1):
        @pl.when(nchunks > w)
        def _prime(w=w):
            _issue(w)

    def _chunk(c, carry):
        p = c % _NBUF

        @pl.when(c + _NBUF - 1 < nchunks)
        def _next():
            _issue(c + _NBUF - 1)

        pltpu.make_async_copy(yh_hbm.at[idx_v.at[pl.ds(0, _K)]], yh_v.at[p],
                              hsem.at[p]).wait()
        pltpu.make_async_copy(y_hbm.at[idx_v.at[pl.ds(0, _K)]], y_v.at[p],
                              ysem.at[p]).wait()

        valid = jnp.minimum(n - c * _K, _K)

        def _row(r, rc):
            @plsc.parallel_loop(0, _NVREG, 1, unroll=_UNROLL, carry=rc)
            def _vblk(j, c2):
                a, cnt = c2
                yv = y_v[p, r, pl.ds(j * 16, 16)]
                hv = yh_v[p, r, pl.ds(j * 16, 16)]
                m = yv != 0.0
                d = jnp.where(m, hv - yv, 0.0)
                return a + d * d, cnt + plsc.all_reduce_population_count(m)

            return _vblk

        return lax.fori_loop(0, valid, _row, carry)

    acc, cf = lax.fori_loop(
        0, nchunks, _chunk,
        (jnp.zeros((16,), jnp.float32), jnp.zeros((16,), jnp.int32)))

    part_v[0, :] = acc
    part_v[1, :] = cf.astype(jnp.float32)
    pltpu.sync_copy(part_v, out_hbm.at[wid])


def _tc_dense(mask_ref, yh_ref, y_ref, out_ref, acc_ref, cnt_ref):
    i = pl.program_id(0)

    @pl.when(i == 0)
    def _init():
        acc_ref[0, 0] = 0.0
        cnt_ref[0, 0] = 0.0

    yh = yh_ref[...]
    yy = y_ref[...]
    w = mask_ref[0]
    nz = yy != 0.0
    d = yh - yy
    rowsum = jnp.sum(jnp.where(nz, d * d, 0.0), axis=1, keepdims=True)
    rowcnt = jnp.sum(jnp.where(nz, 1.0, 0.0), axis=1, keepdims=True)
    dims = (((1,), (0,)), ((), ()))
    acc_ref[0, 0] += jnp.sum(
        lax.dot_general(w, rowsum, dims, preferred_element_type=jnp.float32))
    cnt_ref[0, 0] += jnp.sum(
        lax.dot_general(w, rowcnt, dims, preferred_element_type=jnp.float32))

    @pl.when(i == _TC_GRID - 1)
    def _fin():
        out_ref[0, 0] = acc_ref[0, 0]
        out_ref[0, 1] = cnt_ref[0, 0]


def _tc_final(sc_ref, tc_ref, out_ref):
    p = sc_ref[...]
    acc = jnp.sum(p[:, 0, :]) + tc_ref[0, 0]
    cnt = jnp.sum(p[:, 1, :]) / 16.0 + tc_ref[0, 1]
    out_ref[0, 0] = jnp.sqrt(acc / cnt + _EPS)


def kernel(yhat, y, batch_mask):
    mask_i = batch_mask[:_SPLIT].astype(jnp.int32)
    maskf = batch_mask.astype(jnp.float32).reshape(_ROWS // _TC_BLK, 1, _TC_BLK)
    mesh = plsc.VectorSubcoreMesh(core_axis_name="c", subcore_axis_name="s")
    sc = pl.kernel(
        _sc_body,
        mesh=mesh,
        compiler_params=pltpu.CompilerParams(needs_layout_passes=False),
        out_type=jax.ShapeDtypeStruct((_NW, 2, 16), jnp.float32),
        scratch_types=[
            pltpu.VMEM((_RPW,), jnp.int32),
            pltpu.VMEM((_RPW + 16,), jnp.int32),
            pltpu.VMEM((_NBUF, _K, _COLS), jnp.float32),
            pltpu.VMEM((_NBUF, _K, _COLS), jnp.float32),
            pltpu.VMEM((2, 16), jnp.float32),
            pltpu.SemaphoreType.DMA((_NBUF,)),
            pltpu.SemaphoreType.DMA((_NBUF,)),
        ],
    )
    sc_parts = sc(mask_i, yhat, y)

    base_blk = _SPLIT // _TC_BLK
    tc_parts = pl.pallas_call(
        _tc_dense,
        grid=(_TC_GRID,),
        compiler_params=pltpu.CompilerParams(skip_device_barrier=True),
        in_specs=[
            pl.BlockSpec((1, 1, _TC_BLK), lambda i: (base_blk + i, 0, 0)),
            pl.BlockSpec((_TC_BLK, _COLS), lambda i: (base_blk + i, 0)),
            pl.BlockSpec((_TC_BLK, _COLS), lambda i: (base_blk + i, 0)),
        ],
        out_specs=pl.BlockSpec(memory_space=pltpu.SMEM),
        out_shape=jax.ShapeDtypeStruct((1, 2), jnp.float32),
        scratch_shapes=[
            pltpu.SMEM((1, 1), jnp.float32),
            pltpu.SMEM((1, 1), jnp.float32),
        ],
    )(maskf, yhat, y)

    out = pl.pallas_call(
        _tc_final,
        in_specs=[
            pl.BlockSpec(memory_space=pltpu.VMEM),
            pl.BlockSpec(memory_space=pltpu.SMEM),
        ],
        out_specs=pl.BlockSpec(memory_space=pltpu.SMEM),
        out_shape=jax.ShapeDtypeStruct((1, 1), jnp.float32),
    )(sc_parts, tc_parts)
    return out.reshape(())
